# baseline (speedup 1.0000x reference)
import jax
import jax.numpy as jnp
from jax import lax
from jax.experimental import pallas as pl
from jax.experimental.pallas import tpu as pltpu
from jax.experimental.pallas import tpu_sc as plsc

_BATCH, _HIST, _DIM = 16384, 50, 32
_TOTAL = _BATCH * _HIST
_NW = 32
_BPW = _TOTAL // _NW
_BLK = 128
_KB = 4
_CH = _BLK * _KB
_NCH = _BPW // _CH
_NBLK = _TOTAL // _BLK


def _emb_body(idx_hbm, pt_hbm, it_hbm, out_hbm,
              idx_v, rows_a, rows_b, sg0, sg1, ss0, ss1):
    cid = lax.axis_index("c")
    sid = lax.axis_index("s")
    wid = sid * 2 + cid
    base_blk = wid * (_BPW // _BLK)
    sg = [sg0, sg1]
    ss = [ss0, ss1]

    def fire(ci, slot):
        blk0 = base_blk + ci * _KB
        pltpu.sync_copy(idx_hbm.at[pl.ds(blk0, _KB)], idx_v.at[slot])
        for j in range(_KB):
            pltpu.async_copy(pt_hbm.at[idx_v.at[slot, j]], rows_a.at[slot, j], sg[slot])
            pltpu.async_copy(it_hbm.at[idx_v.at[slot, j]], rows_b.at[slot, j], sg[slot])

    def wait_gathers(slot):
        pltpu.make_async_copy(out_hbm.at[pl.ds(0, _KB)], rows_a.at[slot], sg[slot]).wait()
        pltpu.make_async_copy(out_hbm.at[pl.ds(0, _KB)], rows_b.at[slot], sg[slot]).wait()

    def wait_store(slot):
        pltpu.make_async_copy(rows_a.at[slot], out_hbm.at[pl.ds(0, _KB)], ss[slot]).wait()

    def add_store(ci, slot):
        def addrow(r, c2):
            for j in range(_KB):
                for h in range(2):
                    sl = pl.ds(h * 16, 16)
                    rows_a[slot, j, r, sl] = rows_a[slot, j, r, sl] + rows_b[slot, j, r, sl]
            return c2
        lax.fori_loop(0, _BLK, addrow, 0, unroll=4)
        blk0 = base_blk + ci * _KB
        pltpu.async_copy(rows_a.at[slot], out_hbm.at[pl.ds(blk0, _KB)], ss[slot])

    fire(0, 0)

    def outer(i, carry):
        for b in (0, 1):
            ci = 2 * i + b
            nci = ci + 1
            nslot = 1 - b

            @pl.when(nci < _NCH)
            def _():
                @pl.when(ci >= 1)
                def _():
                    wait_store(nslot)
                fire(nci, nslot)

            wait_gathers(b)
            add_store(ci, b)
        return carry

    lax.fori_loop(0, _NCH // 2, outer, 0)
    wait_store(0)
    wait_store(1)


@jax.jit
def kernel(inputs, pretrain_table, id_table):
    idx = inputs.reshape(_NBLK, _BLK)
    mesh = plsc.VectorSubcoreMesh(core_axis_name="c", subcore_axis_name="s")
    out = pl.kernel(
        _emb_body,
        mesh=mesh,
        out_type=jax.ShapeDtypeStruct((_NBLK, _BLK, _DIM), jnp.float32),
        scratch_types=[
            pltpu.VMEM((2, _KB, _BLK), jnp.int32),
            pltpu.VMEM((2, _KB, _BLK, _DIM), jnp.float32),
            pltpu.VMEM((2, _KB, _BLK, _DIM), jnp.float32),
            pltpu.SemaphoreType.DMA,
            pltpu.SemaphoreType.DMA,
            pltpu.SemaphoreType.DMA,
            pltpu.SemaphoreType.DMA,
        ],
        compiler_params=pltpu.CompilerParams(use_tc_tiling_on_sc=False),
    )(idx, pretrain_table, id_table)
    return out.reshape(_BATCH, _HIST, _DIM)
